# baseline (speedup 1.0000x reference)
import functools

import jax
import jax.numpy as jnp
from jax import lax
from jax.experimental import pallas as pl
from jax.experimental.pallas import tpu as pltpu
from jax.experimental.pallas import tpu_sc as plsc

_B, _C, _N, _K = 4, 128, 512, 16
_NC, _NS = 2, 16
_NW = _NC * _NS
_RPW = (_B * _N) // _NW
_LG = _C // 16


def _stage1_body(x_ref, w_ref, bias_ref, cs_ref, y1_ref, y2_ref):
    b = pl.program_id(0)
    x = x_ref[0]
    g = lax.dot_general(x, x, (((0,), (0,)), ((), ())),
                        preferred_element_type=jnp.float32,
                        precision=lax.Precision.HIGHEST)
    n2 = jnp.sum(x * x, axis=0)
    n2r = n2[None, :]
    n2c = n2[:, None]
    dist = n2c + n2r - 2.0 * g

    iota_row = lax.broadcasted_iota(jnp.int32, (1, _N), 1)
    pn2 = lax.bitcast_convert_type(
        (lax.bitcast_convert_type(n2r, jnp.int32) & ~511) | iota_row,
        jnp.float32)

    pk_cols, m_cols = [], []
    for _ in range(_K):
        m = jnp.min(dist, axis=1, keepdims=True)
        sel = dist == m
        pk = jnp.max(jnp.where(sel, pn2, -jnp.inf), axis=1, keepdims=True)
        dist = jnp.where(sel, jnp.inf, dist)
        pk_cols.append(pk)
        m_cols.append(m)

    pki = lax.bitcast_convert_type(jnp.concatenate(pk_cols, axis=1),
                                   jnp.int32)
    idx_mat = pki & 511
    n2j_mat = lax.bitcast_convert_type(pki & ~511, jnp.float32)
    m_mat = jnp.concatenate(m_cols, axis=1)
    cos_mat = ((n2c + n2j_mat - m_mat) * 0.5) * lax.rsqrt(n2c * n2j_mat)

    e256 = ((lax.broadcasted_iota(jnp.int32, (_K, 256), 1) // 16) ==
            lax.broadcasted_iota(jnp.int32, (_K, 256), 0)).astype(jnp.float32)
    jrep = lax.dot_general(
        idx_mat.astype(jnp.float32), e256, (((1,), (0,)), ((), ())),
        preferred_element_type=jnp.float32,
        precision=lax.Precision.HIGHEST).astype(jnp.int32)
    cos_full = lax.dot_general(
        cos_mat, e256, (((1,), (0,)), ((), ())),
        preferred_element_type=jnp.float32,
        precision=lax.Precision.HIGHEST)
    cs_ref[0] = (lax.bitcast_convert_type(cos_full, jnp.int32) & ~511) | jrep

    w = w_ref[...]
    w1 = w[:, :_C]
    w2 = w[:, _C:]
    y1_ref[0] = lax.dot_general(x, w1, (((0,), (1,)), ((), ())),
                                preferred_element_type=jnp.float32,
                                precision=lax.Precision.HIGHEST)
    y2_ref[0] = lax.dot_general(x, w2, (((0,), (1,)), ((), ())),
                                preferred_element_type=jnp.float32,
                                precision=lax.Precision.HIGHEST) + bias_ref[...]


def _stage1(x, w, bias):
    return pl.pallas_call(
        _stage1_body,
        grid=(_B,),
        in_specs=[
            pl.BlockSpec((1, _C, _N), lambda i: (i, 0, 0)),
            pl.BlockSpec((_C, 2 * _C), lambda i: (0, 0)),
            pl.BlockSpec((1, _C), lambda i: (0, 0)),
        ],
        out_specs=[
            pl.BlockSpec((1, _N, 256), lambda i: (i, 0, 0)),
            pl.BlockSpec((1, _N, _C), lambda i: (i, 0, 0)),
            pl.BlockSpec((1, _N, _C), lambda i: (i, 0, 0)),
        ],
        out_shape=[
            jax.ShapeDtypeStruct((_B, _N, 256), jnp.int32),
            jax.ShapeDtypeStruct((_B, _N, _C), jnp.float32),
            jax.ShapeDtypeStruct((_B, _N, _C), jnp.float32),
        ],
    )(x, w, bias)


_WPB = _NW // _B


def _stage2_body(y1_hbm, y2_hbm, cs_hbm, out_hbm,
                 table_v, cs_v, y2_v, out_v, semt):
    wid = lax.axis_index("s") * _NC + lax.axis_index("c")
    base = wid * _RPW
    batch = wid // _WPB
    tcopy = pltpu.async_copy(y1_hbm.at[pl.ds(batch * _N * _C, _N * _C)],
                             table_v, semt)
    pltpu.sync_copy(cs_hbm.at[pl.ds(base, _RPW)], cs_v)
    pltpu.sync_copy(y2_hbm.at[pl.ds(base, _RPW)], y2_v)
    tcopy.wait()

    u16 = lax.iota(jnp.int32, 16)

    @plsc.parallel_loop(0, _RPW, unroll=1)
    def row_body(r):
        bits = [cs_v[r, pl.ds(k * 16, 16)] for k in range(_K)]
        cks = [lax.bitcast_convert_type(bk & ~511, jnp.float32)
               for bk in bits]
        bvs = [((bk & 511) << 7) + u16 for bk in bits]
        for g in range(_LG):
            sl = pl.ds(g * 16, 16)
            y2g = y2_v[r, sl]
            acc = None
            for k in range(_K):
                val = plsc.load_gather(table_v, [bvs[k] + (g * 16)])
                v = (val + y2g) * cks[k]
                acc = v if acc is None else jnp.maximum(acc, v)
            out_v[r, sl] = acc

    pltpu.sync_copy(out_v, out_hbm.at[pl.ds(base, _RPW)])


@functools.lru_cache(maxsize=1)
def _make_stage2():
    mesh = plsc.VectorSubcoreMesh(
        core_axis_name="c", subcore_axis_name="s",
        num_cores=_NC, num_subcores=_NS)
    return pl.kernel(
        _stage2_body,
        mesh=mesh,
        compiler_params=pltpu.CompilerParams(needs_layout_passes=False),
        out_type=jax.ShapeDtypeStruct((_B * _N, _C), jnp.float32),
        scratch_types=[
            pltpu.VMEM((_N * _C,), jnp.float32),
            pltpu.VMEM((_RPW, 256), jnp.int32),
            pltpu.VMEM((_RPW, _C), jnp.float32),
            pltpu.VMEM((_RPW, _C), jnp.float32),
            pltpu.SemaphoreType.DMA,
        ],
    )


def kernel(x, W, b):
    cs, y1, y2 = _stage1(x, W, b[None, :])
    _stage2 = _make_stage2()
    out_flat = _stage2(
        y1.reshape(_B * _N * _C),
        y2.reshape(_B * _N, _C),
        cs.reshape(_B * _N, 256),
    )
    return jnp.transpose(out_flat.reshape(_B, _N, _C), (0, 2, 1))


# speedup vs baseline: 1.1000x; 1.1000x over previous
import functools

import jax
import jax.numpy as jnp
from jax import lax
from jax.experimental import pallas as pl
from jax.experimental.pallas import tpu as pltpu
from jax.experimental.pallas import tpu_sc as plsc

_B, _C, _N, _K = 4, 128, 512, 16
_NC, _NS = 2, 16
_NW = _NC * _NS
_RPW = (_B * _N) // _NW
_LG = _C // 16


def _stage1_body(x_ref, w_ref, bias_ref, cs_ref, y1_ref, y2_ref):
    b = pl.program_id(0)
    x = x_ref[0]
    g = lax.dot_general(x, x, (((0,), (0,)), ((), ())),
                        preferred_element_type=jnp.float32,
                        precision=lax.Precision.HIGHEST)
    n2 = jnp.sum(x * x, axis=0)
    n2r = n2[None, :]
    n2c = n2[:, None]
    dist = n2c + n2r - 2.0 * g

    iota_row = lax.broadcasted_iota(jnp.int32, (1, _N), 1)
    pn2 = lax.bitcast_convert_type(
        (lax.bitcast_convert_type(n2r, jnp.int32) & ~511) | iota_row,
        jnp.float32)

    pk_cols, m_cols = [], []
    for _ in range(_K):
        m = jnp.min(dist, axis=1, keepdims=True)
        sel = dist == m
        pk = jnp.max(jnp.where(sel, pn2, -jnp.inf), axis=1, keepdims=True)
        dist = jnp.where(sel, jnp.inf, dist)
        pk_cols.append(pk)
        m_cols.append(m)

    pki = lax.bitcast_convert_type(jnp.concatenate(pk_cols, axis=1),
                                   jnp.int32)
    idx_mat = pki & 511
    n2j_mat = lax.bitcast_convert_type(pki & ~511, jnp.float32)
    m_mat = jnp.concatenate(m_cols, axis=1)
    cos_mat = ((n2c + n2j_mat - m_mat) * 0.5) * lax.rsqrt(n2c * n2j_mat)

    e256 = ((lax.broadcasted_iota(jnp.int32, (_K, 256), 1) // 16) ==
            lax.broadcasted_iota(jnp.int32, (_K, 256), 0)).astype(jnp.float32)
    jrep = lax.dot_general(
        idx_mat.astype(jnp.float32), e256, (((1,), (0,)), ((), ())),
        preferred_element_type=jnp.float32,
        precision=lax.Precision.HIGHEST).astype(jnp.int32)
    cos_full = lax.dot_general(
        cos_mat, e256, (((1,), (0,)), ((), ())),
        preferred_element_type=jnp.float32,
        precision=lax.Precision.HIGHEST)
    cs_ref[0] = (lax.bitcast_convert_type(cos_full, jnp.int32) & ~511) | jrep

    w = w_ref[...]
    w1 = w[:, :_C]
    w2 = w[:, _C:]
    y1_ref[0] = lax.dot_general(x, w1, (((0,), (1,)), ((), ())),
                                preferred_element_type=jnp.float32,
                                precision=lax.Precision.HIGHEST)
    y2_ref[0] = lax.dot_general(x, w2, (((0,), (1,)), ((), ())),
                                preferred_element_type=jnp.float32,
                                precision=lax.Precision.HIGHEST) + bias_ref[...]


def _stage1(x, w, bias):
    return pl.pallas_call(
        _stage1_body,
        grid=(_B,),
        in_specs=[
            pl.BlockSpec((1, _C, _N), lambda i: (i, 0, 0)),
            pl.BlockSpec((_C, 2 * _C), lambda i: (0, 0)),
            pl.BlockSpec((1, _C), lambda i: (0, 0)),
        ],
        out_specs=[
            pl.BlockSpec((1, _N, 256), lambda i: (i, 0, 0)),
            pl.BlockSpec((1, _N, _C), lambda i: (i, 0, 0)),
            pl.BlockSpec((1, _N, _C), lambda i: (i, 0, 0)),
        ],
        out_shape=[
            jax.ShapeDtypeStruct((_B, _N, 256), jnp.int32),
            jax.ShapeDtypeStruct((_B, _N, _C), jnp.float32),
            jax.ShapeDtypeStruct((_B, _N, _C), jnp.float32),
        ],
    )(x, w, bias)


_WPB = _NW // _B


def _stage2_body(y1_hbm, y2_hbm, cs_hbm, out_hbm,
                 table_v, cs_v, y2_v, out_v, semt):
    wid = lax.axis_index("s") * _NC + lax.axis_index("c")
    base = wid * _RPW
    batch = wid // _WPB
    tcopy = pltpu.async_copy(y1_hbm.at[pl.ds(batch * _N * _C, _N * _C)],
                             table_v, semt)
    pltpu.sync_copy(cs_hbm.at[pl.ds(base, _RPW)], cs_v)
    pltpu.sync_copy(y2_hbm.at[pl.ds(base, _RPW)], y2_v)
    tcopy.wait()

    u16 = lax.iota(jnp.int32, 16)

    @plsc.parallel_loop(0, _RPW, unroll=2)
    def row_body(r):
        bits = [cs_v[r, pl.ds(k * 16, 16)] for k in range(_K)]
        cks = [lax.bitcast_convert_type(bk & ~511, jnp.float32)
               for bk in bits]
        bvs = [((bk & 511) << 7) + u16 for bk in bits]
        for g in range(_LG):
            sl = pl.ds(g * 16, 16)
            y2g = y2_v[r, sl]
            acc = None
            for k in range(_K):
                val = plsc.load_gather(table_v, [bvs[k] + (g * 16)])
                v = (val + y2g) * cks[k]
                acc = v if acc is None else jnp.maximum(acc, v)
            out_v[r, sl] = acc

    pltpu.sync_copy(out_v, out_hbm.at[pl.ds(base, _RPW)])


@functools.lru_cache(maxsize=1)
def _make_stage2():
    mesh = plsc.VectorSubcoreMesh(
        core_axis_name="c", subcore_axis_name="s",
        num_cores=_NC, num_subcores=_NS)
    return pl.kernel(
        _stage2_body,
        mesh=mesh,
        compiler_params=pltpu.CompilerParams(needs_layout_passes=False),
        out_type=jax.ShapeDtypeStruct((_B * _N, _C), jnp.float32),
        scratch_types=[
            pltpu.VMEM((_N * _C,), jnp.float32),
            pltpu.VMEM((_RPW, 256), jnp.int32),
            pltpu.VMEM((_RPW, _C), jnp.float32),
            pltpu.VMEM((_RPW, _C), jnp.float32),
            pltpu.SemaphoreType.DMA,
        ],
    )


def kernel(x, W, b):
    cs, y1, y2 = _stage1(x, W, b[None, :])
    _stage2 = _make_stage2()
    out_flat = _stage2(
        y1.reshape(_B * _N * _C),
        y2.reshape(_B * _N, _C),
        cs.reshape(_B * _N, 256),
    )
    return jnp.transpose(out_flat.reshape(_B, _N, _C), (0, 2, 1))
